# baseline (speedup 1.0000x reference)
import jax
import jax.numpy as jnp
from jax import lax
from jax.experimental import pallas as pl
from jax.experimental.layout import Layout, with_layout_constraint
from jax.experimental.pallas import tpu as pltpu
from jax.experimental.pallas import tpu_sc as plsc

DIM = 64
NUM_TILES = 32
CHUNK = 200
NBUF = 4


def kernel(x, table):
    full_batch = x.shape[0]
    half = full_batch // 2
    table = with_layout_constraint(
        table, Layout(major_to_minor=(0, 1), tiling=((16,),)))
    out1 = _gather_half(x[:half], table)
    out2 = _gather_half(x[half:], table)
    return jnp.concatenate([out1, out2], axis=0)


def _gather_half(x, table):
    batch, seq = x.shape
    num_idx = batch * seq
    per_tile = num_idx // NUM_TILES
    steps = per_tile // CHUNK
    idx = x.reshape(num_idx)

    mesh = plsc.VectorSubcoreMesh(core_axis_name="c", subcore_axis_name="s")

    @pl.kernel(
        out_type=jax.ShapeDtypeStruct((num_idx, DIM), table.dtype),
        mesh=mesh,
        scratch_types=(
            [pltpu.VMEM((CHUNK,), jnp.int32) for _ in range(NBUF)]
            + [pltpu.VMEM((CHUNK, DIM), jnp.float32) for _ in range(NBUF)]
            + [pltpu.SemaphoreType.DMA for _ in range(NBUF)]
        ),
    )
    def gather_kernel(table_hbm, idx_hbm, out_hbm, *scratch):
        idx_vs = scratch[:NBUF]
        rows_vs = scratch[NBUF:2 * NBUF]
        gsems = scratch[2 * NBUF:]
        wid = lax.axis_index("s") * 2 + lax.axis_index("c")
        tile_base = wid * per_tile

        def out_slice(j):
            return out_hbm.at[pl.ds(tile_base + j * CHUNK, CHUNK)]

        @pl.loop(0, steps // NBUF)
        def _(i):
            j0 = i * NBUF
            descs = []
            for k in range(NBUF):
                base = tile_base + (j0 + k) * CHUNK
                pltpu.sync_copy(idx_hbm.at[pl.ds(base, CHUNK)], idx_vs[k])
                descs.append(pltpu.async_copy(
                    table_hbm.at[idx_vs[k]], rows_vs[k], gsems[k]))
            for k in range(NBUF):
                descs[k].wait()
                pltpu.sync_copy(rows_vs[k], out_slice(j0 + k))

    out = gather_kernel(table, idx)
    return out.reshape(batch, seq, DIM)


# speedup vs baseline: 1.6854x; 1.6854x over previous
import jax
import jax.numpy as jnp
from jax import lax
from jax.experimental import pallas as pl
from jax.experimental.layout import Layout, with_layout_constraint
from jax.experimental.pallas import tpu as pltpu
from jax.experimental.pallas import tpu_sc as plsc

DIM = 64
NUM_TILES = 32
CHUNK = 200
NBUF = 4


def kernel(x, table):
    batch, seq = x.shape
    num_idx = batch * seq
    per_tile = num_idx // NUM_TILES
    steps = per_tile // CHUNK
    idx = jnp.transpose(x).reshape(num_idx)

    table = with_layout_constraint(
        table, Layout(major_to_minor=(0, 1), tiling=((16,),)))

    mesh = plsc.VectorSubcoreMesh(core_axis_name="c", subcore_axis_name="s")

    @pl.kernel(
        out_type=jax.ShapeDtypeStruct((num_idx, DIM), table.dtype),
        mesh=mesh,
        scratch_types=(
            [pltpu.VMEM((CHUNK,), jnp.int32) for _ in range(NBUF)]
            + [pltpu.VMEM((CHUNK, DIM), jnp.float32) for _ in range(NBUF)]
            + [pltpu.SemaphoreType.DMA for _ in range(NBUF)]
        ),
    )
    def gather_kernel(table_hbm, idx_hbm, out_hbm, *scratch):
        idx_vs = scratch[:NBUF]
        rows_vs = scratch[NBUF:2 * NBUF]
        gsems = scratch[2 * NBUF:]
        wid = lax.axis_index("s") * 2 + lax.axis_index("c")
        tile_base = wid * per_tile

        def out_slice(j):
            return out_hbm.at[pl.ds(tile_base + j * CHUNK, CHUNK)]

        @pl.loop(0, steps // NBUF)
        def _(i):
            j0 = i * NBUF
            descs = []
            for k in range(NBUF):
                base = tile_base + (j0 + k) * CHUNK
                pltpu.sync_copy(idx_hbm.at[pl.ds(base, CHUNK)], idx_vs[k])
                descs.append(pltpu.async_copy(
                    table_hbm.at[idx_vs[k]], rows_vs[k], gsems[k]))
            for k in range(NBUF):
                descs[k].wait()
                pltpu.sync_copy(rows_vs[k], out_slice(j0 + k))

    out = gather_kernel(table, idx)
    out = with_layout_constraint(
        out, Layout(major_to_minor=(0, 1), tiling=((8, 128),)))
    return jnp.transpose(out.reshape(seq, batch, DIM), (1, 0, 2))
